# baseline (speedup 1.0000x reference)
import jax
import jax.numpy as jnp
from jax.experimental import pallas as pl
from jax.experimental.pallas import tpu as pltpu

B = 1024
D = 64
K2 = 10
BM = 256


def _bf(x):
    return x.astype(jnp.bfloat16).astype(jnp.float32)


def _mean4(t, a, v, p):
    return 0.25 * _bf(_bf(t) + _bf(a) + _bf(v) + _bf(p))


def _dist_topk_gram_kernel(t_ref, a_ref, v_ref, p_ref,
                           tt_ref, at_ref, vt_ref, pt_ref, s_ref):
    i = pl.program_id(0)
    x = _mean4(t_ref[...], a_ref[...], v_ref[...], p_ref[...])
    xt = _mean4(tt_ref[...], at_ref[...], vt_ref[...], pt_ref[...])
    acc = jnp.zeros((BM, B), jnp.float32)
    for d in range(D):
        acc = acc + jnp.abs(x[:, d:d + 1] - xt[d:d + 1, :])
    lane = jax.lax.broadcasted_iota(jnp.int32, (BM, B), 1)
    row = jax.lax.broadcasted_iota(jnp.int32, (BM, B), 0) + i * BM
    w = (lane == row).astype(jnp.float32)
    dist = acc
    for _ in range(K2):
        m = jnp.max(dist, axis=1, keepdims=True)
        idx = jnp.min(jnp.where(dist == m, lane, B), axis=1, keepdims=True)
        sel = lane == idx
        w = w + sel.astype(jnp.float32)
        dist = jnp.where(sel, -jnp.inf, dist)
    sb = jax.lax.dot_general(w, w, (((0,), (0,)), ((), ())),
                             preferred_element_type=jnp.float32)

    @pl.when(i == 0)
    def _init():
        s_ref[...] = sb

    @pl.when(i != 0)
    def _accum():
        s_ref[...] += sb


def _assemble_tile_kernel(s_ref, out_ref, a_scr):
    i = pl.program_id(0)

    @pl.when(i == 0)
    def _build():
        s = s_ref[...]
        ri = jax.lax.broadcasted_iota(jnp.int32, (B, B), 0)
        ci = jax.lax.broadcasted_iota(jnp.int32, (B, B), 1)
        eye = (ri == ci).astype(jnp.float32)
        inv_rs = 1.0 / (1.0 + jnp.sum(s, axis=1, keepdims=True) / 11.0)
        a_scr[...] = inv_rs * (0.25 * eye + (1.0 / 44.0) * s)

    a = a_scr[...]
    for q in range(4):
        out_ref[:, q * B:(q + 1) * B] = a


def kernel(nodes_t, nodes_a, nodes_v, nodes_p, batch_size):
    del batch_size
    tt = jnp.transpose(nodes_t)
    at = jnp.transpose(nodes_a)
    vt = jnp.transpose(nodes_v)
    pt = jnp.transpose(nodes_p)

    s = pl.pallas_call(
        _dist_topk_gram_kernel,
        grid=(B // BM,),
        in_specs=[
            pl.BlockSpec((BM, D), lambda i: (i, 0)),
            pl.BlockSpec((BM, D), lambda i: (i, 0)),
            pl.BlockSpec((BM, D), lambda i: (i, 0)),
            pl.BlockSpec((BM, D), lambda i: (i, 0)),
            pl.BlockSpec((D, B), lambda i: (0, 0)),
            pl.BlockSpec((D, B), lambda i: (0, 0)),
            pl.BlockSpec((D, B), lambda i: (0, 0)),
            pl.BlockSpec((D, B), lambda i: (0, 0)),
        ],
        out_specs=pl.BlockSpec((B, B), lambda i: (0, 0)),
        out_shape=jax.ShapeDtypeStruct((B, B), jnp.float32),
    )(nodes_t, nodes_a, nodes_v, nodes_p, tt, at, vt, pt)

    adjacency = pl.pallas_call(
        _assemble_tile_kernel,
        grid=(4,),
        in_specs=[pl.BlockSpec((B, B), lambda i: (0, 0))],
        out_specs=pl.BlockSpec((B, 4 * B), lambda i: (i, 0)),
        out_shape=jax.ShapeDtypeStruct((4 * B, 4 * B), jnp.float32),
        scratch_shapes=[pltpu.VMEM((B, B), jnp.float32)],
    )(s)

    nodes_list = jnp.concatenate([nodes_t, nodes_a, nodes_v, nodes_p], axis=0)
    return adjacency, nodes_list


# speedup vs baseline: 1.0287x; 1.0287x over previous
import jax
import jax.numpy as jnp
from jax.experimental import pallas as pl
from jax.experimental.pallas import tpu as pltpu

B = 1024
D = 64
K2 = 10
BM = 256


def _bf(x):
    return x.astype(jnp.bfloat16).astype(jnp.float32)


def _mean4(t, a, v, p):
    return 0.25 * _bf(_bf(t) + _bf(a) + _bf(v) + _bf(p))


def _dist_topk_gram_kernel(t_ref, a_ref, v_ref, p_ref,
                           tt_ref, at_ref, vt_ref, pt_ref, s_ref):
    i = pl.program_id(0)
    x = _mean4(t_ref[...], a_ref[...], v_ref[...], p_ref[...])
    xt = _mean4(tt_ref[...], at_ref[...], vt_ref[...], pt_ref[...])
    acc = jnp.zeros((BM, B), jnp.float32)
    for d in range(D):
        acc = acc + jnp.abs(x[:, d:d + 1] - xt[d:d + 1, :])
    lane = jax.lax.broadcasted_iota(jnp.int32, (BM, B), 1)
    row = jax.lax.broadcasted_iota(jnp.int32, (BM, B), 0) + i * BM
    w = (lane == row).astype(jnp.float32)
    dist = acc
    for _ in range(K2):
        m = jnp.max(dist, axis=1, keepdims=True)
        idx = jnp.min(jnp.where(dist == m, lane, B), axis=1, keepdims=True)
        sel = lane == idx
        w = w + sel.astype(jnp.float32)
        dist = jnp.where(sel, -jnp.inf, dist)
    sb = jax.lax.dot_general(w, w, (((0,), (0,)), ((), ())),
                             preferred_element_type=jnp.float32)

    @pl.when(i == 0)
    def _init():
        s_ref[...] = sb

    @pl.when(i != 0)
    def _accum():
        s_ref[...] += sb


def _assemble_tile_kernel(s_ref, out_ref, a_scr):
    i = pl.program_id(0)
    j = pl.program_id(1)

    @pl.when(jnp.logical_and(i == 0, j == 0))
    def _build():
        s = s_ref[...]
        ri = jax.lax.broadcasted_iota(jnp.int32, (B, B), 0)
        ci = jax.lax.broadcasted_iota(jnp.int32, (B, B), 1)
        eye = (ri == ci).astype(jnp.float32)
        inv_rs = 1.0 / (1.0 + jnp.sum(s, axis=1, keepdims=True) / 11.0)
        a_scr[...] = inv_rs * (0.25 * eye + (1.0 / 44.0) * s)

    out_ref[...] = a_scr[...]


def kernel(nodes_t, nodes_a, nodes_v, nodes_p, batch_size):
    del batch_size
    tt = jnp.transpose(nodes_t)
    at = jnp.transpose(nodes_a)
    vt = jnp.transpose(nodes_v)
    pt = jnp.transpose(nodes_p)

    s = pl.pallas_call(
        _dist_topk_gram_kernel,
        grid=(B // BM,),
        in_specs=[
            pl.BlockSpec((BM, D), lambda i: (i, 0)),
            pl.BlockSpec((BM, D), lambda i: (i, 0)),
            pl.BlockSpec((BM, D), lambda i: (i, 0)),
            pl.BlockSpec((BM, D), lambda i: (i, 0)),
            pl.BlockSpec((D, B), lambda i: (0, 0)),
            pl.BlockSpec((D, B), lambda i: (0, 0)),
            pl.BlockSpec((D, B), lambda i: (0, 0)),
            pl.BlockSpec((D, B), lambda i: (0, 0)),
        ],
        out_specs=pl.BlockSpec((B, B), lambda i: (0, 0)),
        out_shape=jax.ShapeDtypeStruct((B, B), jnp.float32),
    )(nodes_t, nodes_a, nodes_v, nodes_p, tt, at, vt, pt)

    adjacency = pl.pallas_call(
        _assemble_tile_kernel,
        grid=(4, 4),
        in_specs=[pl.BlockSpec((B, B), lambda i, j: (0, 0))],
        out_specs=pl.BlockSpec((B, B), lambda i, j: (i, j)),
        out_shape=jax.ShapeDtypeStruct((4 * B, 4 * B), jnp.float32),
        scratch_shapes=[pltpu.VMEM((B, B), jnp.float32)],
    )(s)

    nodes_list = jnp.concatenate([nodes_t, nodes_a, nodes_v, nodes_p], axis=0)
    return adjacency, nodes_list
